# baseline (speedup 1.0000x reference)
import functools
import math

import jax
import jax.numpy as jnp
from jax.experimental import pallas as pl
from jax.experimental.pallas import tpu as pltpu

H = 16
FH = 64
LSH = 6
NPAN = 8


def _proj_kernel(x_ref, wk_ref, bk_ref, wv_ref, bv_ref, alpha_ref,
                 k_ref, v_ref):
    tb = x_ref.shape[1]
    tt = pl.program_id(1)
    xb = x_ref[0]
    k2 = jax.lax.dot_general(xb, wk_ref[...], (((1,), (1,)), ((), ())),
                             preferred_element_type=jnp.float32) + bk_ref[...]
    v2 = jax.lax.dot_general(xb, wv_ref[...], (((1,), (1,)), ((), ())),
                             preferred_element_type=jnp.float32) + bv_ref[...]
    k3 = k2.reshape(tb, H, FH)
    pos = (tt * tb
           + jax.lax.broadcasted_iota(jnp.int32, (tb, 1, 1), 0)
           ).astype(jnp.float32)
    twof = 2.0 ** jax.lax.broadcasted_iota(
        jnp.int32, (1, 1, FH // 2), 2).astype(jnp.float32)
    alpha = alpha_ref[...].reshape(1, H, FH // 2)
    p = (pos / twof) * alpha
    c = jnp.cos(p)
    s = jnp.sin(p)
    kr = k3[:, :, :FH // 2]
    ki = k3[:, :, FH // 2:]
    k_ref[0] = jnp.concatenate([kr * c - ki * s, kr * s + ki * c], axis=2)
    v_ref[0] = v2.reshape(tb, H, FH)


def _rank_kernel(k_ref, pan_ref, pos_ref, ckc, rpc, ckr, rpr):
    t = k_ref.shape[2]
    rt = 512
    krope = k_ref[0, 0]
    ckc[...] = jnp.zeros((t, 1), jnp.int32)
    rpc[...] = jax.lax.broadcasted_iota(jnp.int32, (t, 1), 0)
    ckr[...] = jnp.zeros((1, t), jnp.int32)
    rpr[...] = jax.lax.broadcasted_iota(jnp.int32, (1, t), 1)
    for i in range(LSH):
        cl = t >> i
        half = cl >> 1
        pi = pan_ref[0, 0, i:i + 1, :]
        s_col = jnp.sum(krope * pi, axis=1, keepdims=True)
        s_row = jnp.transpose(s_col)
        ck_row = ckr[...]
        rp_row = rpr[...]
        acc = jnp.zeros((1, t), jnp.int32)
        for j in range(t // rt):
            sl = pl.ds(j * rt, rt)
            s_c = s_col[j * rt:(j + 1) * rt, :]
            ck_col = ckc[sl, :]
            rp_col = rpc[sl, :]
            less = (s_row < s_c) | ((s_row == s_c) & (rp_row < rp_col))
            m = (less & (ck_row == ck_col)).astype(jnp.int32)
            rank_c = jnp.sum(m, axis=1, keepdims=True)
            acc = acc + jnp.sum(m, axis=0, keepdims=True)
            bit = (rank_c >= half).astype(jnp.int32)
            ckc[sl, :] = ck_col * 2 + bit
            rpc[sl, :] = rank_c - bit * half
        rank_r = (cl - 1) - acc
        bit_r = (rank_r >= half).astype(jnp.int32)
        ckr[...] = ck_row * 2 + bit_r
        rpr[...] = rank_r - bit_r * half
    pos_ref[0, 0] = ckr[...] * FH + rpr[...]


def _attn_kernel(k_ref, v_ref, pos_ref, o_ref):
    t = k_ref.shape[2]
    pw = 128
    kv = jnp.concatenate([k_ref[0, 0], v_ref[0, 0]], axis=1)
    pos_row = pos_ref[0, 0]
    scale = 1.0 / math.sqrt(FH)
    r = jax.lax.broadcasted_iota(jnp.int32, (pw, pw), 0)
    cc = jax.lax.broadcasted_iota(jnp.int32, (pw, pw), 1)
    allowed = ((r // FH) == (cc // FH)) & (r != cc)
    neg = jnp.float32(-jnp.inf)
    for j in range(t // pw):
        p_iota = j * pw + jax.lax.broadcasted_iota(jnp.int32, (pw, 1), 0)
        onehot = (pos_row == p_iota).astype(jnp.float32)
        kvb = jax.lax.dot_general(onehot, kv, (((1,), (0,)), ((), ())),
                                  preferred_element_type=jnp.float32)
        kb = kvb[:, :FH]
        vb = kvb[:, FH:]
        a = jax.lax.dot_general(kb, kb, (((1,), (1,)), ((), ())),
                                preferred_element_type=jnp.float32)
        a = jnp.where(allowed, a * scale, neg)
        a = a - jnp.max(a, axis=1, keepdims=True)
        e = jnp.exp(a)
        w = e / jnp.sum(e, axis=1, keepdims=True)
        o_ref[0, 0, pl.ds(j * pw, pw), :] = jax.lax.dot_general(
            w, vb, (((1,), (0,)), ((), ())),
            preferred_element_type=jnp.float32)


def kernel(x, mask, W_K, b_K, W_V, b_V, rope_alpha):
    B, T, C = x.shape
    del mask
    panels = jax.random.normal(jax.random.key(42), (LSH, B * H, 1, 1, FH),
                               dtype=jnp.float32)[:, :, 0, 0, :]
    pan = jnp.transpose(panels, (1, 0, 2)).reshape(B, H, LSH, FH)
    pan = jnp.pad(pan, ((0, 0), (0, 0), (0, NPAN - LSH), (0, 0)))

    tb = 256
    k_arr, v_arr = pl.pallas_call(
        _proj_kernel,
        grid=(B, T // tb),
        in_specs=[
            pl.BlockSpec((1, tb, C), lambda b, t: (b, t, 0)),
            pl.BlockSpec((C, C), lambda b, t: (0, 0)),
            pl.BlockSpec((1, C), lambda b, t: (0, 0)),
            pl.BlockSpec((C, C), lambda b, t: (0, 0)),
            pl.BlockSpec((1, C), lambda b, t: (0, 0)),
            pl.BlockSpec((1, H * FH // 2), lambda b, t: (0, 0)),
        ],
        out_specs=[
            pl.BlockSpec((1, tb, H, FH), lambda b, t: (b, t, 0, 0)),
            pl.BlockSpec((1, tb, H, FH), lambda b, t: (b, t, 0, 0)),
        ],
        out_shape=[
            jax.ShapeDtypeStruct((B, T, H, FH), jnp.float32),
            jax.ShapeDtypeStruct((B, T, H, FH), jnp.float32),
        ],
    )(x, W_K, b_K.reshape(1, -1), W_V, b_V.reshape(1, -1),
      rope_alpha.reshape(1, -1))

    k_bh = jnp.transpose(k_arr, (0, 2, 1, 3))
    v_bh = jnp.transpose(v_arr, (0, 2, 1, 3))

    pos = pl.pallas_call(
        _rank_kernel,
        grid=(B, H),
        in_specs=[
            pl.BlockSpec((1, 1, T, FH), lambda b, h: (b, h, 0, 0)),
            pl.BlockSpec((1, 1, NPAN, FH), lambda b, h: (b, h, 0, 0)),
        ],
        out_specs=pl.BlockSpec((1, 1, 1, T), lambda b, h: (b, h, 0, 0)),
        out_shape=jax.ShapeDtypeStruct((B, H, 1, T), jnp.int32),
        scratch_shapes=[
            pltpu.VMEM((T, 1), jnp.int32),
            pltpu.VMEM((T, 1), jnp.int32),
            pltpu.VMEM((1, T), jnp.int32),
            pltpu.VMEM((1, T), jnp.int32),
        ],
    )(k_bh, pan)

    o_arr = pl.pallas_call(
        _attn_kernel,
        grid=(B, H),
        in_specs=[
            pl.BlockSpec((1, 1, T, FH), lambda b, h: (b, h, 0, 0)),
            pl.BlockSpec((1, 1, T, FH), lambda b, h: (b, h, 0, 0)),
            pl.BlockSpec((1, 1, 1, T), lambda b, h: (b, h, 0, 0)),
        ],
        out_specs=pl.BlockSpec((1, 1, T, FH), lambda b, h: (b, h, 0, 0)),
        out_shape=jax.ShapeDtypeStruct((B, H, T, FH), jnp.float32),
    )(k_bh, v_bh, pos)

    return jnp.transpose(o_arr, (0, 2, 1, 3)).reshape(B, T, H * FH)


# speedup vs baseline: 1.0459x; 1.0459x over previous
import functools
import math

import jax
import jax.numpy as jnp
from jax import lax
from jax.experimental import pallas as pl
from jax.experimental.pallas import tpu as pltpu
from jax.experimental.pallas import tpu_sc as plsc

H = 16
FH = 64
LSH = 6
NPAN = 8


def _proj_kernel(x_ref, wk_ref, bk_ref, wv_ref, bv_ref, alpha_ref,
                 kv_ref):
    tb = x_ref.shape[1]
    tt = pl.program_id(1)
    xb = x_ref[0]
    k2 = jax.lax.dot_general(xb, wk_ref[...], (((1,), (1,)), ((), ())),
                             preferred_element_type=jnp.float32) + bk_ref[...]
    v2 = jax.lax.dot_general(xb, wv_ref[...], (((1,), (1,)), ((), ())),
                             preferred_element_type=jnp.float32) + bv_ref[...]
    k3 = k2.reshape(tb, H, FH)
    pos = (tt * tb
           + jax.lax.broadcasted_iota(jnp.int32, (tb, 1, 1), 0)
           ).astype(jnp.float32)
    twof = 2.0 ** jax.lax.broadcasted_iota(
        jnp.int32, (1, 1, FH // 2), 2).astype(jnp.float32)
    alpha = alpha_ref[...].reshape(1, H, FH // 2)
    p = (pos / twof) * alpha
    c = jnp.cos(p)
    s = jnp.sin(p)
    kr = k3[:, :, :FH // 2]
    ki = k3[:, :, FH // 2:]
    kv_ref[0] = jnp.concatenate(
        [kr * c - ki * s, kr * s + ki * c, v2.reshape(tb, H, FH)], axis=2)


def _rank_kernel(k_ref, pan_ref, pos_ref, ckc, rpc, ckr, rpr):
    t = k_ref.shape[2]
    rt = 512
    krope = k_ref[0, 0, :, :FH]
    ckc[...] = jnp.zeros((t, 1), jnp.int32)
    rpc[...] = jax.lax.broadcasted_iota(jnp.int32, (t, 1), 0)
    ckr[...] = jnp.zeros((1, t), jnp.int32)
    rpr[...] = jax.lax.broadcasted_iota(jnp.int32, (1, t), 1)
    for i in range(LSH):
        cl = t >> i
        half = cl >> 1
        pi = pan_ref[0, 0, i:i + 1, :]
        s_col = jnp.sum(krope * pi, axis=1, keepdims=True)
        s_row = jnp.transpose(s_col)
        ck_row = ckr[...]
        rp_row = rpr[...]
        acc = jnp.zeros((1, t), jnp.int32)
        for j in range(t // rt):
            sl = pl.ds(j * rt, rt)
            s_c = s_col[j * rt:(j + 1) * rt, :]
            ck_col = ckc[sl, :]
            rp_col = rpc[sl, :]
            less = (s_row < s_c) | ((s_row == s_c) & (rp_row < rp_col))
            m = (less & (ck_row == ck_col)).astype(jnp.int32)
            rank_c = jnp.sum(m, axis=1, keepdims=True)
            acc = acc + jnp.sum(m, axis=0, keepdims=True)
            bit = (rank_c >= half).astype(jnp.int32)
            ckc[sl, :] = ck_col * 2 + bit
            rpc[sl, :] = rank_c - bit * half
        rank_r = (cl - 1) - acc
        bit_r = (rank_r >= half).astype(jnp.int32)
        ckr[...] = ck_row * 2 + bit_r
        rpr[...] = rank_r - bit_r * half
    pos_ref[0, 0] = ckr[...] * FH + rpr[...]


def _permute_sc_body(kv_hbm, pos_hbm, kvb_hbm, posv, idxg, bkv, sem):
    t = posv.shape[0]
    iw = 128
    ch = bkv.shape[0]
    sub = ch // iw
    wid = lax.axis_index("s") * 2 + lax.axis_index("c")
    pltpu.sync_copy(pos_hbm.at[wid], posv)
    base = wid * t

    for j in range(t // 16):
        c = (j * 16) // iw
        q = (j * 16) % iw
        idxg[c, pl.ds(q, 16)] = posv[pl.ds(j * 16, 16)] + base

    for c in range(t // ch):
        pltpu.sync_copy(kv_hbm.at[pl.ds(base + c * ch, ch)], bkv)
        for q in range(sub):
            pltpu.async_copy(bkv.at[pl.ds(q * iw, iw)],
                             kvb_hbm.at[idxg.at[c * sub + q]], sem).wait()


def _attn_kernel(kv_ref, o_ref):
    t = kv_ref.shape[2]
    pw = 128
    scale = 1.0 / math.sqrt(FH)
    r = jax.lax.broadcasted_iota(jnp.int32, (pw, pw), 0)
    cc = jax.lax.broadcasted_iota(jnp.int32, (pw, pw), 1)
    allowed = ((r // FH) == (cc // FH)) & (r != cc)
    neg = jnp.float32(-jnp.inf)
    for j in range(t // pw):
        sl = pl.ds(j * pw, pw)
        kb = kv_ref[0, 0, sl, :FH]
        vb = kv_ref[0, 0, sl, FH:]
        a = jax.lax.dot_general(kb, kb, (((1,), (1,)), ((), ())),
                                preferred_element_type=jnp.float32)
        a = jnp.where(allowed, a * scale, neg)
        a = a - jnp.max(a, axis=1, keepdims=True)
        e = jnp.exp(a)
        w = e / jnp.sum(e, axis=1, keepdims=True)
        o_ref[0, 0, sl, :] = jax.lax.dot_general(
            w, vb, (((1,), (0,)), ((), ())),
            preferred_element_type=jnp.float32)


def kernel(x, mask, W_K, b_K, W_V, b_V, rope_alpha):
    B, T, C = x.shape
    del mask
    panels = jax.random.normal(jax.random.key(42), (LSH, B * H, 1, 1, FH),
                               dtype=jnp.float32)[:, :, 0, 0, :]
    pan = jnp.transpose(panels, (1, 0, 2)).reshape(B, H, LSH, FH)
    pan = jnp.pad(pan, ((0, 0), (0, 0), (0, NPAN - LSH), (0, 0)))

    tb = 256
    kv_arr = pl.pallas_call(
        _proj_kernel,
        grid=(B, T // tb),
        in_specs=[
            pl.BlockSpec((1, tb, C), lambda b, t: (b, t, 0)),
            pl.BlockSpec((C, C), lambda b, t: (0, 0)),
            pl.BlockSpec((1, C), lambda b, t: (0, 0)),
            pl.BlockSpec((C, C), lambda b, t: (0, 0)),
            pl.BlockSpec((1, C), lambda b, t: (0, 0)),
            pl.BlockSpec((1, H * FH // 2), lambda b, t: (0, 0)),
        ],
        out_specs=pl.BlockSpec((1, tb, H, 2 * FH), lambda b, t: (b, t, 0, 0)),
        out_shape=jax.ShapeDtypeStruct((B, T, H, 2 * FH), jnp.float32),
    )(x, W_K, b_K.reshape(1, -1), W_V, b_V.reshape(1, -1),
      rope_alpha.reshape(1, -1))

    kv_bh = jnp.transpose(kv_arr, (0, 2, 1, 3))

    pos = pl.pallas_call(
        _rank_kernel,
        grid=(B, H),
        in_specs=[
            pl.BlockSpec((1, 1, T, 2 * FH), lambda b, h: (b, h, 0, 0)),
            pl.BlockSpec((1, 1, NPAN, FH), lambda b, h: (b, h, 0, 0)),
        ],
        out_specs=pl.BlockSpec((1, 1, 1, T), lambda b, h: (b, h, 0, 0)),
        out_shape=jax.ShapeDtypeStruct((B, H, 1, T), jnp.int32),
        scratch_shapes=[
            pltpu.VMEM((T, 1), jnp.int32),
            pltpu.VMEM((T, 1), jnp.int32),
            pltpu.VMEM((1, T), jnp.int32),
            pltpu.VMEM((1, T), jnp.int32),
        ],
    )(kv_bh, pan)

    ch = 512
    mesh = plsc.VectorSubcoreMesh(core_axis_name="c", subcore_axis_name="s")
    permute = functools.partial(
        pl.kernel, mesh=mesh,
        out_type=jax.ShapeDtypeStruct((B * H * T, 2 * FH), jnp.float32),
        scratch_types=[
            pltpu.VMEM((T,), jnp.int32),
            pltpu.VMEM((T // 128, 128), jnp.int32),
            pltpu.VMEM((ch, 2 * FH), jnp.float32),
            pltpu.SemaphoreType.DMA,
        ],
    )(_permute_sc_body)
    kvb_flat = permute(kv_bh.reshape(B * H * T, 2 * FH),
                       pos.reshape(B * H, T))
    kvb = kvb_flat.reshape(B, H, T, 2 * FH)

    o_arr = pl.pallas_call(
        _attn_kernel,
        grid=(B, H),
        in_specs=[
            pl.BlockSpec((1, 1, T, 2 * FH), lambda b, h: (b, h, 0, 0)),
        ],
        out_specs=pl.BlockSpec((1, 1, T, FH), lambda b, h: (b, h, 0, 0)),
        out_shape=jax.ShapeDtypeStruct((B, H, T, FH), jnp.float32),
    )(kvb)

    return jnp.transpose(o_arr, (0, 2, 1, 3)).reshape(B, T, H * FH)


# speedup vs baseline: 1.8954x; 1.8122x over previous
import functools
import math

import jax
import jax.numpy as jnp
from jax import lax
from jax.experimental import pallas as pl
from jax.experimental.pallas import tpu as pltpu
from jax.experimental.pallas import tpu_sc as plsc

H = 16
FH = 64
LSH = 6
NPAN = 8


def _proj_kernel(x_ref, wk_ref, bk_ref, wv_ref, bv_ref, alpha_ref,
                 kv_ref):
    tb = x_ref.shape[1]
    tt = pl.program_id(1)
    xb = x_ref[0]
    k2 = jax.lax.dot_general(xb, wk_ref[...], (((1,), (1,)), ((), ())),
                             preferred_element_type=jnp.float32) + bk_ref[...]
    v2 = jax.lax.dot_general(xb, wv_ref[...], (((1,), (1,)), ((), ())),
                             preferred_element_type=jnp.float32) + bv_ref[...]
    k3 = k2.reshape(tb, H, FH)
    pos = (tt * tb
           + jax.lax.broadcasted_iota(jnp.int32, (tb, 1, 1), 0)
           ).astype(jnp.float32)
    twof = 2.0 ** jax.lax.broadcasted_iota(
        jnp.int32, (1, 1, FH // 2), 2).astype(jnp.float32)
    alpha = alpha_ref[...].reshape(1, H, FH // 2)
    p = (pos / twof) * alpha
    c = jnp.cos(p)
    s = jnp.sin(p)
    kr = k3[:, :, :FH // 2]
    ki = k3[:, :, FH // 2:]
    kv_ref[0] = jnp.concatenate(
        [kr * c - ki * s, kr * s + ki * c, v2.reshape(tb, H, FH)], axis=2)


def _rank_kernel(k_ref, pan_ref, pos_ref, ckc, rpc, ckr, rpr):
    t = k_ref.shape[2]
    rt = 512
    krope = k_ref[0, 0, :, :FH]
    ckc[...] = jnp.zeros((t, 1), jnp.int32)
    rpc[...] = jax.lax.broadcasted_iota(jnp.int32, (t, 1), 0)
    ckr[...] = jnp.zeros((1, t), jnp.int32)
    rpr[...] = jax.lax.broadcasted_iota(jnp.int32, (1, t), 1)
    for i in range(LSH):
        cl = t >> i
        half = cl >> 1
        pi = pan_ref[0, 0, i:i + 1, :]
        s_col = jnp.sum(krope * pi, axis=1, keepdims=True)
        s_row = jnp.transpose(s_col)
        ck_row = ckr[...]
        rp_row = rpr[...]
        acc = jnp.zeros((1, t), jnp.int32)
        for j in range(t // rt):
            sl = pl.ds(j * rt, rt)
            s_c = s_col[j * rt:(j + 1) * rt, :]
            ck_col = ckc[sl, :]
            rp_col = rpc[sl, :]
            less = (s_row < s_c) | ((s_row == s_c) & (rp_row < rp_col))
            m = (less & (ck_row == ck_col)).astype(jnp.int32)
            rank_c = jnp.sum(m, axis=1, keepdims=True)
            acc = acc + jnp.sum(m, axis=0, keepdims=True)
            bit = (rank_c >= half).astype(jnp.int32)
            ckc[sl, :] = ck_col * 2 + bit
            rpc[sl, :] = rank_c - bit * half
        rank_r = (cl - 1) - acc
        bit_r = (rank_r >= half).astype(jnp.int32)
        ckr[...] = ck_row * 2 + bit_r
        rpr[...] = rank_r - bit_r * half
    pos_ref[0, 0] = ckr[...] * FH + rpr[...]


def _permute_sc_body(kv_hbm, pos_hbm, kvb_hbm, posv, idxg, bkv, sem):
    t = posv.shape[0]
    iw = 128
    ch = bkv.shape[0]
    sub = ch // iw
    nslab = pos_hbm.shape[0]
    rep = 32 // nslab
    wid = lax.axis_index("s") * 2 + lax.axis_index("c")
    slab = wid // rep
    part = wid % rep
    nch = t // ch // rep
    pltpu.sync_copy(pos_hbm.at[slab], posv)
    base = slab * t

    tok0 = part * (t // rep)
    nj = t // 16 // rep
    for jj in range(nj):
        idxg[(jj * 16) // iw, pl.ds((jj * 16) % iw, 16)] = (
            posv[pl.ds(tok0 + jj * 16, 16)] + base)

    for cc in range(nch):
        pltpu.sync_copy(kv_hbm.at[pl.ds(base + tok0 + cc * ch, ch)], bkv)
        for q in range(sub):
            pltpu.async_copy(bkv.at[pl.ds(q * iw, iw)],
                             kvb_hbm.at[idxg.at[cc * sub + q]], sem).wait()


def _attn_kernel(kv_ref, o_ref):
    t = kv_ref.shape[2]
    pw = 128
    scale = 1.0 / math.sqrt(FH)
    r = jax.lax.broadcasted_iota(jnp.int32, (pw, pw), 0)
    cc = jax.lax.broadcasted_iota(jnp.int32, (pw, pw), 1)
    allowed = ((r // FH) == (cc // FH)) & (r != cc)
    neg = jnp.float32(-jnp.inf)
    for j in range(t // pw):
        sl = pl.ds(j * pw, pw)
        kb = kv_ref[0, 0, sl, :FH]
        vb = kv_ref[0, 0, sl, FH:]
        a = jax.lax.dot_general(kb, kb, (((1,), (1,)), ((), ())),
                                preferred_element_type=jnp.float32)
        a = jnp.where(allowed, a * scale, neg)
        a = a - jnp.max(a, axis=1, keepdims=True)
        e = jnp.exp(a)
        w = e / jnp.sum(e, axis=1, keepdims=True)
        o_ref[0, 0, sl, :] = jax.lax.dot_general(
            w, vb, (((1,), (0,)), ((), ())),
            preferred_element_type=jnp.float32)


def _pipeline(x, W_K, b_K, W_V, b_V, rope_alpha, pan):
    B, T, C = x.shape
    tb = 256
    kv_arr = pl.pallas_call(
        _proj_kernel,
        grid=(B, T // tb),
        in_specs=[
            pl.BlockSpec((1, tb, C), lambda b, t: (b, t, 0)),
            pl.BlockSpec((C, C), lambda b, t: (0, 0)),
            pl.BlockSpec((1, C), lambda b, t: (0, 0)),
            pl.BlockSpec((C, C), lambda b, t: (0, 0)),
            pl.BlockSpec((1, C), lambda b, t: (0, 0)),
            pl.BlockSpec((1, H * FH // 2), lambda b, t: (0, 0)),
        ],
        out_specs=pl.BlockSpec((1, tb, H, 2 * FH), lambda b, t: (b, t, 0, 0)),
        out_shape=jax.ShapeDtypeStruct((B, T, H, 2 * FH), jnp.float32),
    )(x, W_K, b_K.reshape(1, -1), W_V, b_V.reshape(1, -1),
      rope_alpha.reshape(1, -1))

    kv_bh = jnp.transpose(kv_arr, (0, 2, 1, 3))

    pos = pl.pallas_call(
        _rank_kernel,
        grid=(B, H),
        in_specs=[
            pl.BlockSpec((1, 1, T, 2 * FH), lambda b, h: (b, h, 0, 0)),
            pl.BlockSpec((1, 1, NPAN, FH), lambda b, h: (b, h, 0, 0)),
        ],
        out_specs=pl.BlockSpec((1, 1, 1, T), lambda b, h: (b, h, 0, 0)),
        out_shape=jax.ShapeDtypeStruct((B, H, 1, T), jnp.int32),
        scratch_shapes=[
            pltpu.VMEM((T, 1), jnp.int32),
            pltpu.VMEM((T, 1), jnp.int32),
            pltpu.VMEM((1, T), jnp.int32),
            pltpu.VMEM((1, T), jnp.int32),
        ],
    )(kv_bh, pan)

    ch = 512
    mesh = plsc.VectorSubcoreMesh(core_axis_name="c", subcore_axis_name="s")
    permute = functools.partial(
        pl.kernel, mesh=mesh,
        out_type=jax.ShapeDtypeStruct((B * H * T, 2 * FH), jnp.float32),
        scratch_types=[
            pltpu.VMEM((T,), jnp.int32),
            pltpu.VMEM((T // 128, 128), jnp.int32),
            pltpu.VMEM((ch, 2 * FH), jnp.float32),
            pltpu.SemaphoreType.DMA,
        ],
    )(_permute_sc_body)
    kvb_flat = permute(kv_bh.reshape(B * H * T, 2 * FH),
                       pos.reshape(B * H, T))
    kvb = kvb_flat.reshape(B, H, T, 2 * FH)

    o_arr = pl.pallas_call(
        _attn_kernel,
        grid=(B, H),
        in_specs=[
            pl.BlockSpec((1, 1, T, 2 * FH), lambda b, h: (b, h, 0, 0)),
        ],
        out_specs=pl.BlockSpec((1, 1, T, FH), lambda b, h: (b, h, 0, 0)),
        out_shape=jax.ShapeDtypeStruct((B, H, T, FH), jnp.float32),
    )(kvb)

    return jnp.transpose(o_arr, (0, 2, 1, 3)).reshape(B, T, H * FH)


def kernel(x, mask, W_K, b_K, W_V, b_V, rope_alpha):
    B, T, C = x.shape
    del mask
    panels = jax.random.normal(jax.random.key(42), (LSH, B * H, 1, 1, FH),
                               dtype=jnp.float32)[:, :, 0, 0, :]
    pan = jnp.transpose(panels, (1, 0, 2)).reshape(B, H, LSH, FH)
    pan = jnp.pad(pan, ((0, 0), (0, 0), (0, NPAN - LSH), (0, 0)))

    devs = jax.devices()
    if len(devs) >= 2 and B % 2 == 0:
        import numpy as np
        from jax.sharding import PartitionSpec as P
        mesh = jax.sharding.Mesh(np.asarray(devs[:2]), ("d",))
        smap = jax.shard_map(
            _pipeline, mesh=mesh,
            in_specs=(P("d"), P(), P(), P(), P(), P(), P("d")),
            out_specs=P("d"),
            check_vma=False,
        )
        return smap(x, W_K, b_K, W_V, b_V, rope_alpha, pan)
    return _pipeline(x, W_K, b_K, W_V, b_V, rope_alpha, pan)


# speedup vs baseline: 2.1047x; 1.1104x over previous
import functools
import math

import jax
import jax.numpy as jnp
from jax import lax
from jax.experimental import pallas as pl
from jax.experimental.pallas import tpu as pltpu
from jax.experimental.pallas import tpu_sc as plsc

H = 16
FH = 64
LSH = 6
NPAN = 8


def _proj_kernel(x_ref, wk_ref, bk_ref, wv_ref, bv_ref, alpha_ref,
                 kv_ref):
    tb = x_ref.shape[1]
    tt = pl.program_id(1)
    xb = x_ref[0]
    k2 = jax.lax.dot_general(xb, wk_ref[...], (((1,), (1,)), ((), ())),
                             preferred_element_type=jnp.float32) + bk_ref[...]
    v2 = jax.lax.dot_general(xb, wv_ref[...], (((1,), (1,)), ((), ())),
                             preferred_element_type=jnp.float32) + bv_ref[...]
    k3 = k2.reshape(tb, H, FH)
    pos = (tt * tb
           + jax.lax.broadcasted_iota(jnp.int32, (tb, 1, 1), 0)
           ).astype(jnp.float32)
    twof = 2.0 ** jax.lax.broadcasted_iota(
        jnp.int32, (1, 1, FH // 2), 2).astype(jnp.float32)
    alpha = alpha_ref[...].reshape(1, H, FH // 2)
    p = (pos / twof) * alpha
    c = jnp.cos(p)
    s = jnp.sin(p)
    kr = k3[:, :, :FH // 2]
    ki = k3[:, :, FH // 2:]
    kv_ref[0] = jnp.concatenate(
        [kr * c - ki * s, kr * s + ki * c, v2.reshape(tb, H, FH)], axis=2)


def _rank_kernel(k_ref, pan_ref, pos_ref, ckc, rpc):
    t = k_ref.shape[2]
    rt = 512
    krope = k_ref[0, 0, :, :FH]
    ckc[...] = jnp.zeros((t, 1), jnp.int32)
    rpc[...] = jax.lax.broadcasted_iota(jnp.int32, (t, 1), 0)
    for i in range(LSH):
        cl = t >> i
        half = cl >> 1
        pi = pan_ref[0, 0, i:i + 1, :]
        s_col = jnp.sum(krope * pi, axis=1, keepdims=True)
        s_row = jnp.transpose(s_col)
        ck_row = jnp.transpose(ckc[...])
        rp_row = jnp.transpose(rpc[...])
        for j in range(t // rt):
            sl = pl.ds(j * rt, rt)
            s_c = s_col[j * rt:(j + 1) * rt, :]
            ck_col = ckc[sl, :]
            rp_col = rpc[sl, :]
            less = (s_row < s_c) | ((s_row == s_c) & (rp_row < rp_col))
            m = (less & (ck_row == ck_col)).astype(jnp.int32)
            rank_c = jnp.sum(m, axis=1, keepdims=True)
            bit = (rank_c >= half).astype(jnp.int32)
            ckc[sl, :] = ck_col * 2 + bit
            rpc[sl, :] = rank_c - bit * half
    pos_ref[0, 0] = jnp.transpose(ckc[...] * FH + rpc[...])


def _permute_sc_body(kv_hbm, pos_hbm, kvb_hbm, posv, idxg, bkv, sem):
    t = posv.shape[0]
    iw = 128
    ch = bkv.shape[0]
    sub = ch // iw
    nslab = pos_hbm.shape[0]
    rep = 32 // nslab
    wid = lax.axis_index("s") * 2 + lax.axis_index("c")
    slab = wid // rep
    part = wid % rep
    nch = t // ch // rep
    pltpu.sync_copy(pos_hbm.at[slab], posv)
    base = slab * t

    tok0 = part * (t // rep)
    nj = t // 16 // rep
    for jj in range(nj):
        idxg[(jj * 16) // iw, pl.ds((jj * 16) % iw, 16)] = (
            posv[pl.ds(tok0 + jj * 16, 16)] + base)

    for cc in range(nch):
        pltpu.sync_copy(kv_hbm.at[pl.ds(base + tok0 + cc * ch, ch)], bkv)
        for q in range(sub):
            pltpu.async_copy(bkv.at[pl.ds(q * iw, iw)],
                             kvb_hbm.at[idxg.at[cc * sub + q]], sem).wait()


def _attn_kernel(kv_ref, o_ref):
    t = kv_ref.shape[2]
    pw = 128
    scale = 1.0 / math.sqrt(FH)
    r = jax.lax.broadcasted_iota(jnp.int32, (pw, pw), 0)
    cc = jax.lax.broadcasted_iota(jnp.int32, (pw, pw), 1)
    allowed = ((r // FH) == (cc // FH)) & (r != cc)
    neg = jnp.float32(-jnp.inf)
    for j in range(t // pw):
        sl = pl.ds(j * pw, pw)
        kb = kv_ref[0, 0, sl, :FH]
        vb = kv_ref[0, 0, sl, FH:]
        a = jax.lax.dot_general(kb, kb, (((1,), (1,)), ((), ())),
                                preferred_element_type=jnp.float32)
        a = jnp.where(allowed, a * scale, neg)
        a = a - jnp.max(a, axis=1, keepdims=True)
        e = jnp.exp(a)
        w = e / jnp.sum(e, axis=1, keepdims=True)
        o_ref[0, 0, sl, :] = jax.lax.dot_general(
            w, vb, (((1,), (0,)), ((), ())),
            preferred_element_type=jnp.float32)


def _pipeline(x, W_K, b_K, W_V, b_V, rope_alpha, pan):
    B, T, C = x.shape
    tb = 256
    kv_arr = pl.pallas_call(
        _proj_kernel,
        grid=(B, T // tb),
        in_specs=[
            pl.BlockSpec((1, tb, C), lambda b, t: (b, t, 0)),
            pl.BlockSpec((C, C), lambda b, t: (0, 0)),
            pl.BlockSpec((1, C), lambda b, t: (0, 0)),
            pl.BlockSpec((C, C), lambda b, t: (0, 0)),
            pl.BlockSpec((1, C), lambda b, t: (0, 0)),
            pl.BlockSpec((1, H * FH // 2), lambda b, t: (0, 0)),
        ],
        out_specs=pl.BlockSpec((1, tb, H, 2 * FH), lambda b, t: (b, t, 0, 0)),
        out_shape=jax.ShapeDtypeStruct((B, T, H, 2 * FH), jnp.float32),
    )(x, W_K, b_K.reshape(1, -1), W_V, b_V.reshape(1, -1),
      rope_alpha.reshape(1, -1))

    kv_bh = jnp.transpose(kv_arr, (0, 2, 1, 3))

    pos = pl.pallas_call(
        _rank_kernel,
        grid=(B, H),
        in_specs=[
            pl.BlockSpec((1, 1, T, 2 * FH), lambda b, h: (b, h, 0, 0)),
            pl.BlockSpec((1, 1, NPAN, FH), lambda b, h: (b, h, 0, 0)),
        ],
        out_specs=pl.BlockSpec((1, 1, 1, T), lambda b, h: (b, h, 0, 0)),
        out_shape=jax.ShapeDtypeStruct((B, H, 1, T), jnp.int32),
        scratch_shapes=[
            pltpu.VMEM((T, 1), jnp.int32),
            pltpu.VMEM((T, 1), jnp.int32),
        ],
    )(kv_bh, pan)

    ch = 512
    mesh = plsc.VectorSubcoreMesh(core_axis_name="c", subcore_axis_name="s")
    permute = functools.partial(
        pl.kernel, mesh=mesh,
        out_type=jax.ShapeDtypeStruct((B * H * T, 2 * FH), jnp.float32),
        scratch_types=[
            pltpu.VMEM((T,), jnp.int32),
            pltpu.VMEM((T // 128, 128), jnp.int32),
            pltpu.VMEM((ch, 2 * FH), jnp.float32),
            pltpu.SemaphoreType.DMA,
        ],
    )(_permute_sc_body)
    kvb_flat = permute(kv_bh.reshape(B * H * T, 2 * FH),
                       pos.reshape(B * H, T))
    kvb = kvb_flat.reshape(B, H, T, 2 * FH)

    o_arr = pl.pallas_call(
        _attn_kernel,
        grid=(B, H),
        in_specs=[
            pl.BlockSpec((1, 1, T, 2 * FH), lambda b, h: (b, h, 0, 0)),
        ],
        out_specs=pl.BlockSpec((1, 1, T, FH), lambda b, h: (b, h, 0, 0)),
        out_shape=jax.ShapeDtypeStruct((B, H, T, FH), jnp.float32),
    )(kvb)

    return jnp.transpose(o_arr, (0, 2, 1, 3)).reshape(B, T, H * FH)


def kernel(x, mask, W_K, b_K, W_V, b_V, rope_alpha):
    B, T, C = x.shape
    del mask
    panels = jax.random.normal(jax.random.key(42), (LSH, B * H, 1, 1, FH),
                               dtype=jnp.float32)[:, :, 0, 0, :]
    pan = jnp.transpose(panels, (1, 0, 2)).reshape(B, H, LSH, FH)
    pan = jnp.pad(pan, ((0, 0), (0, 0), (0, NPAN - LSH), (0, 0)))

    devs = jax.devices()
    if len(devs) >= 2 and B % 2 == 0:
        import numpy as np
        from jax.sharding import PartitionSpec as P
        mesh = jax.sharding.Mesh(np.asarray(devs[:2]), ("d",))
        smap = jax.shard_map(
            _pipeline, mesh=mesh,
            in_specs=(P("d"), P(), P(), P(), P(), P(), P("d")),
            out_specs=P("d"),
            check_vma=False,
        )
        return smap(x, W_K, b_K, W_V, b_V, rope_alpha, pan)
    return _pipeline(x, W_K, b_K, W_V, b_V, rope_alpha, pan)
